# baseline (speedup 1.0000x reference)
import functools

import jax
import jax.numpy as jnp
from jax import lax
from jax.experimental import pallas as pl
from jax.experimental.pallas import tpu as pltpu
from jax.experimental.pallas import tpu_sc as plsc

EPS = 1e-07
B, C, H, W = 16, 4, 512, 512
HW = H * W
NSLAB = B * C

SC_SLABS = 16
TC_SLABS = NSLAB - SC_SLABS
NSTREAM = 8
STEPS = TC_SLABS // NSTREAM

TILES = 32
TPS = TILES // SC_SLABS
SC_ROWS = H // TPS
CB_ROWS = 16
CB = CB_ROWS * W
NCH = SC_ROWS // CB_ROWS


def _sc_body(x_hbm, t_hbm, out_hbm, xb0, xb1, tb0, tb1, ri_v, rc_v,
             sx0, sx1, st0, st1):
    nc = 2
    wid = lax.axis_index("s") * nc + lax.axis_index("c")
    slab = wid // TPS
    row0 = (wid % TPS) * SC_ROWS

    def start(ch, bufs, sems):
        r = row0 + ch * CB_ROWS
        hx = pltpu.make_async_copy(
            x_hbm.at[slab, pl.ds(r, CB_ROWS), :], bufs[0], sems[0])
        ht = pltpu.make_async_copy(
            t_hbm.at[slab, pl.ds(r, CB_ROWS), :], bufs[1], sems[1])
        hx.start()
        ht.start()
        return hx, ht

    bufs = ((xb0, tb0), (xb1, tb1))
    sems = ((sx0, st0), (sx1, st1))
    pending = start(0, bufs[0], sems[0])
    acc_i = jnp.zeros((16,), jnp.float32)
    acc_c = jnp.zeros((16,), jnp.float32)
    for ch in range(NCH):
        cur = bufs[ch % 2]
        hx, ht = pending
        if ch + 1 < NCH:
            nxt = start(ch + 1, bufs[(ch + 1) % 2], sems[(ch + 1) % 2])
        hx.wait()
        ht.wait()

        def body(j, carry):
            ai, ac = carry
            r = j // (W // 16)
            c = (j % (W // 16)) * 16
            xv = cur[0][r, pl.ds(c, 16)]
            tv = cur[1][r, pl.ds(c, 16)]
            return ai + xv * tv, ac + (xv + tv)

        acc_i, acc_c = lax.fori_loop(0, CB // 16, body, (acc_i, acc_c),
                                     unroll=8)
        if ch + 1 < NCH:
            pending = nxt
    ri_v[...] = acc_i
    rc_v[...] = acc_c
    pltpu.sync_copy(ri_v, out_hbm.at[wid, 0])
    pltpu.sync_copy(rc_v, out_hbm.at[wid, 1])


_sc_partial = functools.partial(
    pl.kernel,
    mesh=plsc.VectorSubcoreMesh(core_axis_name="c", subcore_axis_name="s"),
    out_type=jax.ShapeDtypeStruct((TILES, 2, 16), jnp.float32),
    scratch_types=[
        pltpu.VMEM((CB_ROWS, W), jnp.float32),
        pltpu.VMEM((CB_ROWS, W), jnp.float32),
        pltpu.VMEM((CB_ROWS, W), jnp.float32),
        pltpu.VMEM((CB_ROWS, W), jnp.float32),
        pltpu.VMEM((16,), jnp.float32),
        pltpu.VMEM((16,), jnp.float32),
        pltpu.SemaphoreType.DMA,
        pltpu.SemaphoreType.DMA,
        pltpu.SemaphoreType.DMA,
        pltpu.SemaphoreType.DMA,
    ],
)(_sc_body)


def _tc_kernel(*refs):
    pair_refs = refs[:2 * NSTREAM]
    out_ref = refs[2 * NSTREAM]
    i = pl.program_id(0)

    @pl.when(i == 0)
    def _init():
        out_ref[...] = jnp.zeros_like(out_ref)

    row = jax.lax.broadcasted_iota(jnp.int32, (B, C), 0)
    col = jax.lax.broadcasted_iota(jnp.int32, (B, C), 1)
    acc_i = jnp.zeros((B, C), jnp.float32)
    acc_c = jnp.zeros((B, C), jnp.float32)
    for q in range(NSTREAM):
        xq = pair_refs[2 * q][0]
        tq = pair_refs[2 * q + 1][0]
        slab = i + SC_SLABS + q * STEPS
        hot = (row == slab // C) & (col == slab % C)
        acc_i += jnp.where(hot, jnp.sum(xq * tq), 0.0)
        acc_c += jnp.where(hot, jnp.sum(xq + tq), 0.0)
    out_ref[0] += acc_i
    out_ref[1] += acc_c


def _combine_kernel(dom_ref, tc_ref, sc_ref, out_ref):
    inter = tc_ref[0]
    card = tc_ref[1]
    row = jax.lax.broadcasted_iota(jnp.int32, (B, C), 0)
    col = jax.lax.broadcasted_iota(jnp.int32, (B, C), 1)
    for s in range(SC_SLABS):
        hot = (row == s // C) & (col == s % C)
        inter += jnp.where(hot, jnp.sum(sc_ref[pl.ds(TPS * s, TPS), 0, :]), 0.0)
        card += jnp.where(hot, jnp.sum(sc_ref[pl.ds(TPS * s, TPS), 1, :]), 0.0)
    d0 = dom_ref[:, 0:1]
    d1 = dom_ref[:, 1:2]
    w1 = (d1 > d0).astype(jnp.float32)
    w0 = 1.0 - w1
    i0 = jnp.sum(inter * w0, axis=0, keepdims=True)
    c0 = jnp.sum(card * w0, axis=0, keepdims=True)
    i1 = jnp.sum(inter * w1, axis=0, keepdims=True)
    c1 = jnp.sum(card * w1, axis=0, keepdims=True)
    loss0 = 1.0 - jnp.mean(2.0 * i0 / (c0 + EPS))
    loss1 = 1.0 - jnp.mean(2.0 * i1 / (c1 + EPS))
    lane = jax.lax.broadcasted_iota(jnp.int32, (1, 4), 1)
    out_ref[...] = jnp.where(
        lane == 0, loss0 + loss1, jnp.where(lane == 1, loss0, loss1)
    )


def kernel(x, label_true, domains):
    xr = x.reshape(NSLAB, H, W)
    tr = label_true.reshape(NSLAB, H, W)
    sc_out = _sc_partial(xr, tr)
    specs = []
    operands = []
    for q in range(NSTREAM):
        specs.append(
            pl.BlockSpec((1, H, W), lambda i, q=q: (i + SC_SLABS + q * STEPS, 0, 0)))
        specs.append(
            pl.BlockSpec((1, H, W), lambda i, q=q: (i + SC_SLABS + q * STEPS, 0, 0)))
        operands.append(xr)
        operands.append(tr)
    tc_maps = pl.pallas_call(
        _tc_kernel,
        grid=(STEPS,),
        in_specs=specs,
        out_specs=pl.BlockSpec((2, B, C), lambda i: (0, 0, 0)),
        out_shape=jax.ShapeDtypeStruct((2, B, C), jnp.float32),
    )(*operands)

    out = pl.pallas_call(
        _combine_kernel,
        out_shape=jax.ShapeDtypeStruct((1, 4), jnp.float32),
    )(domains, tc_maps, sc_out)
    return (out[0, 0], (out[0, 1], out[0, 2]))


# speedup vs baseline: 1.3447x; 1.3447x over previous
import jax
import jax.numpy as jnp
from jax.experimental import pallas as pl
from jax.experimental.pallas import tpu as pltpu

EPS = 1e-07
B, C, H, W = 16, 4, 512, 512
SPLIT = 2
NROW = B * C * SPLIT
NSTREAM = 16
STEPS = NROW // NSTREAM


def _dice_kernel(*refs):
    dom_ref = refs[0]
    pair_refs = refs[1:1 + 2 * NSTREAM]
    out_ref = refs[1 + 2 * NSTREAM]
    acc_ref = refs[2 + 2 * NSTREAM]
    i = pl.program_id(0)
    n = pl.num_programs(0)

    @pl.when(i == 0)
    def _init():
        acc_ref[...] = jnp.zeros_like(acc_ref)

    row = jax.lax.broadcasted_iota(jnp.int32, (B, C), 0)
    col = jax.lax.broadcasted_iota(jnp.int32, (B, C), 1)
    acc_i = jnp.zeros((B, C), jnp.float32)
    acc_c = jnp.zeros((B, C), jnp.float32)
    for q in range(NSTREAM):
        xq = pair_refs[2 * q][0]
        tq = pair_refs[2 * q + 1][0]
        slab = (i + q * STEPS) // SPLIT
        hot = (row == slab // C) & (col == slab % C)
        acc_i += jnp.where(hot, jnp.sum(xq * tq), 0.0)
        acc_c += jnp.where(hot, jnp.sum(xq + tq), 0.0)
    acc_ref[0] += acc_i
    acc_ref[1] += acc_c

    @pl.when(i == n - 1)
    def _epilogue():
        inter = acc_ref[0]
        card = acc_ref[1]
        d0 = dom_ref[:, 0:1]
        d1 = dom_ref[:, 1:2]
        w1 = (d1 > d0).astype(jnp.float32)
        w0 = 1.0 - w1
        i0 = jnp.sum(inter * w0, axis=0, keepdims=True)
        c0 = jnp.sum(card * w0, axis=0, keepdims=True)
        i1 = jnp.sum(inter * w1, axis=0, keepdims=True)
        c1 = jnp.sum(card * w1, axis=0, keepdims=True)
        loss0 = 1.0 - jnp.mean(2.0 * i0 / (c0 + EPS))
        loss1 = 1.0 - jnp.mean(2.0 * i1 / (c1 + EPS))
        lane = jax.lax.broadcasted_iota(jnp.int32, (1, 4), 1)
        out_ref[...] = jnp.where(
            lane == 0, loss0 + loss1, jnp.where(lane == 1, loss0, loss1)
        )


def kernel(x, label_true, domains):
    xr = x.reshape(NROW, H // SPLIT, W)
    tr = label_true.reshape(NROW, H // SPLIT, W)
    specs = [pl.BlockSpec((B, 2), lambda i: (0, 0))]
    operands = [domains]
    for q in range(NSTREAM):
        specs.append(
            pl.BlockSpec((1, H // SPLIT, W), lambda i, q=q: (i + q * STEPS, 0, 0)))
        specs.append(
            pl.BlockSpec((1, H // SPLIT, W), lambda i, q=q: (i + q * STEPS, 0, 0)))
        operands.append(xr)
        operands.append(tr)
    out = pl.pallas_call(
        _dice_kernel,
        grid=(STEPS,),
        in_specs=specs,
        out_specs=pl.BlockSpec((1, 4), lambda i: (0, 0)),
        out_shape=jax.ShapeDtypeStruct((1, 4), jnp.float32),
        scratch_shapes=[pltpu.VMEM((2, B, C), jnp.float32)],
    )(*operands)
    return (out[0, 0], (out[0, 1], out[0, 2]))


# speedup vs baseline: 1.3616x; 1.0126x over previous
import jax
import jax.numpy as jnp
from jax.experimental import pallas as pl
from jax.experimental.pallas import tpu as pltpu

EPS = 1e-07
B, C, H, W = 16, 4, 512, 512
NSTREAM = 8
STEPS = (B * C) // NSTREAM


def _dice_kernel(*refs):
    dom_ref = refs[0]
    pair_refs = refs[1:1 + 2 * NSTREAM]
    out_ref = refs[1 + 2 * NSTREAM]
    acc_ref = refs[2 + 2 * NSTREAM]
    i = pl.program_id(0)
    n = pl.num_programs(0)

    @pl.when(i == 0)
    def _init():
        acc_ref[...] = jnp.zeros_like(acc_ref)

    row = jax.lax.broadcasted_iota(jnp.int32, (B, C), 0)
    col = jax.lax.broadcasted_iota(jnp.int32, (B, C), 1)
    acc_i = jnp.zeros((B, C), jnp.float32)
    acc_c = jnp.zeros((B, C), jnp.float32)
    for q in range(NSTREAM):
        xq = pair_refs[2 * q][0]
        tq = pair_refs[2 * q + 1][0]
        slab = i + q * STEPS
        hot = (row == slab // C) & (col == slab % C)
        acc_i += jnp.where(hot, jnp.sum(xq * tq), 0.0)
        acc_c += jnp.where(hot, jnp.sum(xq + tq), 0.0)
    acc_ref[0] += acc_i
    acc_ref[1] += acc_c

    @pl.when(i == n - 1)
    def _epilogue():
        inter = acc_ref[0]
        card = acc_ref[1]
        d0 = dom_ref[:, 0:1]
        d1 = dom_ref[:, 1:2]
        w1 = (d1 > d0).astype(jnp.float32)
        w0 = 1.0 - w1
        i0 = jnp.sum(inter * w0, axis=0, keepdims=True)
        c0 = jnp.sum(card * w0, axis=0, keepdims=True)
        i1 = jnp.sum(inter * w1, axis=0, keepdims=True)
        c1 = jnp.sum(card * w1, axis=0, keepdims=True)
        loss0 = 1.0 - jnp.mean(2.0 * i0 / (c0 + EPS))
        loss1 = 1.0 - jnp.mean(2.0 * i1 / (c1 + EPS))
        lane = jax.lax.broadcasted_iota(jnp.int32, (1, 4), 1)
        out_ref[...] = jnp.where(
            lane == 0, loss0 + loss1, jnp.where(lane == 1, loss0, loss1)
        )


def kernel(x, label_true, domains):
    xr = x.reshape(B * C, H, W)
    tr = label_true.reshape(B * C, H, W)
    specs = [pl.BlockSpec((B, 2), lambda i: (0, 0))]
    operands = [domains]
    for q in range(NSTREAM):
        specs.append(pl.BlockSpec((1, H, W), lambda i, q=q: (i + q * STEPS, 0, 0)))
        specs.append(pl.BlockSpec((1, H, W), lambda i, q=q: (i + q * STEPS, 0, 0)))
        operands.append(xr)
        operands.append(tr)
    out = pl.pallas_call(
        _dice_kernel,
        grid=(STEPS,),
        in_specs=specs,
        out_specs=pl.BlockSpec((1, 4), lambda i: (0, 0)),
        out_shape=jax.ShapeDtypeStruct((1, 4), jnp.float32),
        scratch_shapes=[pltpu.VMEM((2, B, C), jnp.float32)],
    )(*operands)
    return (out[0, 0], (out[0, 1], out[0, 2]))
